# baseline (speedup 1.0000x reference)
import functools

import jax
import jax.numpy as jnp
from jax import lax
from jax.experimental import pallas as pl
from jax.experimental.pallas import tpu as pltpu
from jax.experimental.pallas import tpu_sc as plsc

B = 16384
D = 1024
NR = 51
NEG = -1e30
TBL = B + 8
TRASH = 3 * TBL
NMID = 2 * B
NTAIL = 3 * B
NAUG = NMID + NTAIL

_ROWS1 = 1024


def _tc1_body(x_ref, lab_ref, w_ref, bias_ref, g_ref, cls_ref):
    x = x_ref[...]
    w = w_ref[...]
    p = lax.dot_general(x, w, (((1,), (0,)), ((), ())),
                        preferred_element_type=jnp.float32,
                        precision=lax.Precision.HIGHEST)
    p = p + bias_ref[...]
    lane = lax.broadcasted_iota(jnp.int32, (_ROWS1, 128), 1)
    valid = lane < NR
    lm = jnp.where(valid, p, NEG)
    mx = jnp.max(lm, axis=1, keepdims=True)
    e = jnp.where(valid, jnp.exp(p - mx), 0.0)
    s = jnp.sum(e, axis=1, keepdims=True)
    prob = e / s
    pmax = jnp.max(prob, axis=1, keepdims=True)
    top = jnp.max(jnp.where((prob == pmax) & valid, lane, -1),
                  axis=1, keepdims=True)
    labv = lab_ref[...]
    labeled = labv != 0
    midf = (top >= 17) & (top < 34) & (pmax > 0.5) & (~labeled)
    tailf = (top >= 34) & (pmax > 0.3) & (~labeled)
    cls = jnp.where(labeled, 1, jnp.where(midf, 2, jnp.where(tailf, 3, 0)))
    g = jnp.where(lane >= 64 + NR, NEG, p)
    lab_f = lax.bitcast_convert_type(labv, jnp.float32)
    top_f = lax.bitcast_convert_type(top, jnp.float32)
    g = jnp.where(lane == 124, lab_f, g)
    g = jnp.where(lane == 125, top_f, g)
    g_ref[...] = g
    cls_ref[...] = cls


def _tc1(feat, lab2, wt, bias):
    return pl.pallas_call(
        _tc1_body,
        grid=(B // _ROWS1,),
        in_specs=[
            pl.BlockSpec((_ROWS1, D), lambda i: (i, 0)),
            pl.BlockSpec((_ROWS1, 1), lambda i: (i, 0)),
            pl.BlockSpec((D, 128), lambda i: (0, 0)),
            pl.BlockSpec((1, 128), lambda i: (0, 0)),
        ],
        out_specs=[
            pl.BlockSpec((_ROWS1, 128), lambda i: (i, 0)),
            pl.BlockSpec((_ROWS1, 1), lambda i: (i, 0)),
        ],
        out_shape=[
            jax.ShapeDtypeStruct((B, 128), jnp.float32),
            jax.ShapeDtypeStruct((B, 1), jnp.int32),
        ],
    )(feat, lab2, wt, bias)


_RPT = B // 32


def _vec_scalars16(v):
    return [v[l] for l in range(16)]


def _sca1_body(meta_hbm, tcnt_hbm, meta_v, cvec_v):
    cid = lax.axis_index("c")
    sid = lax.axis_index("s")
    wid = sid * 2 + cid
    iota = lax.broadcasted_iota(jnp.int32, (16,), 0)
    pltpu.sync_copy(meta_hbm.at[pl.ds(wid * _RPT, _RPT)], meta_v)

    def cnt_step(j, carry):
        co, cm, ct = carry
        cls = meta_v[pl.ds(j * 16, 16)]
        for c in _vec_scalars16(cls):
            co = co + (c == 1).astype(jnp.int32)
            cm = cm + (c == 2).astype(jnp.int32)
            ct = ct + (c == 3).astype(jnp.int32)
        return co, cm, ct

    co, cm, ct = lax.fori_loop(0, _RPT // 16, cnt_step, (0, 0, 0))
    cvec = jnp.where(iota == 0, co,
                     jnp.where(iota == 1, cm,
                               jnp.where(iota == 2, ct, 0)))
    cvec_v[pl.ds(0, 16)] = cvec
    pltpu.sync_copy(cvec_v, tcnt_hbm.at[wid])


def _sca1(meta):
    mesh = plsc.VectorSubcoreMesh(core_axis_name="c", subcore_axis_name="s")
    f = pl.kernel(
        _sca1_body,
        out_type=jax.ShapeDtypeStruct((32, 128), jnp.int32),
        mesh=mesh,
        scratch_types=[
            pltpu.VMEM((_RPT,), jnp.int32),
            pltpu.VMEM((128,), jnp.int32),
        ],
    )
    return f(meta)


def _sca2_body(g_hbm, meta_hbm, tcnt_hbm, gall_hbm, cnt_hbm,
               rows_v, meta_v, cvec_v, allc_v,
               io0, io1, io2, io3, sem):
    io_refs = [io0, io1, io2, io3]
    cid = lax.axis_index("c")
    sid = lax.axis_index("s")
    wid = sid * 2 + cid
    iota = lax.broadcasted_iota(jnp.int32, (16,), 0)
    pltpu.sync_copy(meta_hbm.at[pl.ds(wid * _RPT, _RPT)], meta_v)
    pltpu.sync_copy(tcnt_hbm, allc_v)

    bo, bm, bt = 0, 0, 0
    to, tm, tt = 0, 0, 0
    for t in range(32):
        v = allc_v[t, pl.ds(0, 16)]
        o, m, tl = v[0], v[1], v[2]
        pred = (t < wid).astype(jnp.int32)
        bo = bo + pred * o
        bm = bm + pred * m
        bt = bt + pred * tl
        to = to + o
        tm = tm + m
        tt = tt + tl

    descs = [None, None]
    for ch in range(_RPT // 128):
        def jj_step(jj, carry):
            bo, bm, bt = carry
            cls = meta_v[pl.ds(ch * 128 + jj * 16, 16)]
            dvec = jnp.full((16,), TRASH, jnp.int32)
            for l, c in enumerate(_vec_scalars16(cls)):
                iso = (c == 1).astype(jnp.int32)
                ism = (c == 2).astype(jnp.int32)
                ist = (c == 3).astype(jnp.int32)
                dest = jnp.where(
                    c == 1, bo,
                    jnp.where(c == 2, TBL + bm,
                              jnp.where(c == 3, 2 * TBL + bt, TRASH)))
                dvec = jnp.where(iota == l, dest, dvec)
                bo = bo + iso
                bm = bm + ism
                bt = bt + ist
            io_refs[ch][pl.ds(jj * 16, 16)] = dvec
            return bo, bm, bt

        bo, bm, bt = lax.fori_loop(0, 8, jj_step, (bo, bm, bt))
        buf = ch % 2
        if descs[buf] is not None:
            descs[buf].wait()
        pltpu.sync_copy(g_hbm.at[pl.ds(wid * _RPT + ch * 128, 128)],
                        rows_v.at[buf])
        descs[buf] = pltpu.async_copy(
            rows_v.at[buf], gall_hbm.at[io_refs[ch]], sem)
    for dsc in descs:
        if dsc is not None:
            dsc.wait()

    @pl.when(wid == 0)
    def _emit_counts():
        cvec_v[pl.ds(0, 16)] = jnp.where(
            iota == 0, to,
            jnp.where(iota == 1, tm, jnp.where(iota == 2, tt, 0)))
        pltpu.sync_copy(cvec_v.at[pl.ds(0, 16)], cnt_hbm)


def _sca2(g, meta, tcnt):
    mesh = plsc.VectorSubcoreMesh(core_axis_name="c", subcore_axis_name="s")
    f = pl.kernel(
        _sca2_body,
        out_type=(
            jax.ShapeDtypeStruct((3 * TBL + 8, 128), jnp.float32),
            jax.ShapeDtypeStruct((16,), jnp.int32),
        ),
        mesh=mesh,
        scratch_types=[
            pltpu.VMEM((2, 128, 128), jnp.float32),
            pltpu.VMEM((_RPT,), jnp.int32),
            pltpu.VMEM((128,), jnp.int32),
            pltpu.VMEM((32, 128), jnp.int32),
        ] + [pltpu.VMEM((128,), jnp.int32) for _ in range(4)] + [
            pltpu.SemaphoreType.DMA,
        ],
    )
    return f(g, meta, tcnt)


def _sca(g, meta):
    tcnt = _sca1(meta)
    return _sca2(g, meta, tcnt)


def _rotl(x, d):
    return (x << jnp.uint32(d)) | (x >> jnp.uint32(32 - d))


def _threefry(k0, k1, x0, x1):
    rotations = ((13, 15, 26, 6), (17, 29, 16, 24))
    ks2 = k0 ^ k1 ^ jnp.uint32(0x1BD11BDA)
    ks = (k0, k1, ks2)
    x = [x0 + ks[0], x1 + ks[1]]
    for i in range(5):
        for r in rotations[i % 2]:
            x[0] = x[0] + x[1]
            x[1] = _rotl(x[1], r)
            x[1] = x[0] ^ x[1]
        x[0] = x[0] + ks[(i + 1) % 3]
        x[1] = x[1] + ks[(i + 2) % 3] + jnp.uint32(i + 1)
    return x[0], x[1]


def _tc0_body(cnt_ref, key_ref, idx_ref, *, partitionable):
    n_o = cnt_ref[0]
    span = jnp.maximum(n_o, 1).astype(jnp.uint32)
    m_mid = cnt_ref[1]
    m_tail = cnt_ref[2]
    r = lax.broadcasted_iota(jnp.int32, (NAUG // 128, 128), 0)
    c = lax.broadcasted_iota(jnp.int32, (NAUG // 128, 128), 1)
    flat = r * 128 + c
    part = flat < NMID
    i_in = jnp.where(part, flat, flat - NMID).astype(jnp.uint32)

    def keysel(a, b):
        return jnp.where(part, key_ref[a], key_ref[b])

    def bits(k0, k1):
        if partitionable:
            y0, y1 = _threefry(k0, k1, jnp.zeros_like(i_in), i_in)
            return y0 ^ y1
        n32 = jnp.where(part, 2 * m_mid, 3 * m_tail).astype(jnp.uint32)
        half = (n32 + jnp.uint32(1)) // jnp.uint32(2)
        hi_lane = i_in >= half
        j = jnp.where(hi_lane, i_in - half, i_in)
        x0 = j
        x1 = j + half
        x1 = jnp.where(x1 < n32, x1, jnp.uint32(0))
        y0, y1 = _threefry(k0, k1, x0, x1)
        return jnp.where(hi_lane, y1, y0)

    hi = bits(keysel(0, 4), keysel(1, 5))
    lo = bits(keysel(2, 6), keysel(3, 7))
    mult = jnp.remainder(jnp.uint32(2 ** 16), span)
    mult = jnp.remainder(mult * mult, span)
    off = jnp.remainder(hi, span) * mult + jnp.remainder(lo, span)
    off = jnp.remainder(off, span)
    idx_ref[...] = off.astype(jnp.int32)


def _tc0(counts, keys):
    body = functools.partial(
        _tc0_body, partitionable=bool(jax.config.jax_threefry_partitionable))
    return pl.pallas_call(
        body,
        in_specs=[
            pl.BlockSpec(memory_space=pltpu.SMEM),
            pl.BlockSpec(memory_space=pltpu.SMEM),
        ],
        out_specs=pl.BlockSpec((NAUG // 128, 128), lambda: (0, 0)),
        out_shape=jax.ShapeDtypeStruct((NAUG // 128, 128), jnp.int32),
    )(counts, keys)


_RPW = NAUG // 32
_CH = 128
_NSPL = 4


def _scb_body(idx_hbm, gall_hbm, cnt_hbm, oo_hbm, om_hbm,
              idxo0, idxo1, idxm0, idxm1, obuf, mbuf, cvec_v, semg, semw):
    idxo = [idxo0, idxo1]
    idxm = [idxm0, idxm1]
    cid = lax.axis_index("c")
    sid = lax.axis_index("s")
    nc = 2
    wid = sid * nc + cid
    iota = lax.broadcasted_iota(jnp.int32, (16,), 0)
    pltpu.sync_copy(cnt_hbm, cvec_v)
    cvec = cvec_v[...]
    m_mid = cvec[1]
    m_tail = cvec[2]

    for c in range(_RPW // _CH):
        buf = c % 2
        i0 = wid * _RPW + c * _CH
        gchunk = i0 // _CH
        valid_chunk = jnp.where(i0 < NMID, i0 < 2 * m_mid,
                                i0 - NMID < 3 * m_tail)

        @pl.when(valid_chunk)
        def _do_chunk(i0=i0, gchunk=gchunk, buf=buf):
            pltpu.sync_copy(idx_hbm.at[gchunk], idxo[buf])
            for l in range(8):
                i = i0 + l * 16 + iota
                part = i < NMID
                jm = jnp.where(i < m_mid, i, i - m_mid)
                jm = jnp.where(i < 2 * m_mid, jm, 0)
                it = i - NMID
                jt = jnp.where(it >= m_tail, it - m_tail, it)
                jt = jnp.where(jt >= m_tail, jt - m_tail, jt)
                jt = jnp.where(it < 3 * m_tail, jt, 0)
                j = jnp.where(part, TBL + jm, 2 * TBL + jt)
                idxm[buf][pl.ds(l * 16, 16)] = j
            gd = []
            spl = _CH // _NSPL
            for s in range(_NSPL):
                gd.append(pltpu.async_copy(
                    gall_hbm.at[idxo[buf].at[pl.ds(s * spl, spl)]],
                    obuf.at[buf, pl.ds(s * spl, spl)], semg))
            gd.append(pltpu.async_copy(
                gall_hbm.at[idxm[buf]], mbuf.at[buf], semg))
            for dsc in gd:
                dsc.wait()
            pltpu.sync_copy(obuf.at[buf], oo_hbm.at[pl.ds(i0, _CH)])
            pltpu.sync_copy(mbuf.at[buf], om_hbm.at[pl.ds(i0, _CH)])

    _ = semw


def _scb(idx3, gall, counts):
    mesh = plsc.VectorSubcoreMesh(core_axis_name="c", subcore_axis_name="s")
    f = pl.kernel(
        _scb_body,
        out_type=(
            jax.ShapeDtypeStruct((NAUG, 128), jnp.float32),
            jax.ShapeDtypeStruct((NAUG, 128), jnp.float32),
        ),
        mesh=mesh,
        scratch_types=[
            pltpu.VMEM((128,), jnp.int32),
            pltpu.VMEM((128,), jnp.int32),
            pltpu.VMEM((128,), jnp.int32),
            pltpu.VMEM((128,), jnp.int32),
            pltpu.VMEM((2, _CH, 128), jnp.float32),
            pltpu.VMEM((2, _CH, 128), jnp.float32),
            pltpu.VMEM((16,), jnp.int32),
            pltpu.SemaphoreType.DMA,
            pltpu.SemaphoreType.DMA,
        ],
    )
    return f(idx3, gall, counts)


_ROWS2 = 2048
_NBLK2 = NAUG // _ROWS2


def _tc2_body(cnt_ref, o_ref, m_ref, out_ref):
    pid = pl.program_id(0)
    m_mid = cnt_ref[1]
    m_tail = cnt_ref[2]
    o = o_ref[...]
    m = m_ref[...]
    olab = lax.bitcast_convert_type(o[:, 124:125], jnp.int32)
    mlab = lax.bitcast_convert_type(m[:, 125:126], jnp.int32)
    lane = lax.broadcasted_iota(jnp.int32, (_ROWS2, 128), 1)
    valid = (lane >= 64) & (lane < 64 + NR)
    lg = 0.7 * o + 0.3 * m
    lm = jnp.where(valid, lg, NEG)
    mx = jnp.max(lm, axis=1, keepdims=True)
    e = jnp.where(valid, jnp.exp(lg - mx), 0.0)
    s = jnp.sum(e, axis=1, keepdims=True)
    lse = mx + jnp.log(s)
    pick = (0.7 * jnp.sum(jnp.where(lane - 64 == olab, lg, 0.0), axis=1,
                          keepdims=True)
            + 0.3 * jnp.sum(jnp.where(lane - 64 == mlab, lg, 0.0), axis=1,
                            keepdims=True))
    rl = lse - pick
    rowid = pid * _ROWS2 + lax.broadcasted_iota(jnp.int32, (_ROWS2, 1), 0)
    limit = jnp.where(rowid < NMID, 2 * m_mid, NMID + 3 * m_tail)
    part_sum = jnp.sum(jnp.where(rowid < limit, rl, 0.0))

    @pl.when(pid == 0)
    def _init():
        out_ref[...] = jnp.zeros((1, 1), jnp.float32)

    out_ref[...] = out_ref[...] + part_sum

    @pl.when(pid == _NBLK2 - 1)
    def _fin():
        total = (2 * m_mid + 3 * m_tail).astype(jnp.float32)
        out_ref[...] = jnp.where(total > 0, out_ref[...] / total,
                                 jnp.zeros((1, 1), jnp.float32))


def _tc2(counts, out_o, out_m):
    return pl.pallas_call(
        _tc2_body,
        grid=(_NBLK2,),
        in_specs=[
            pl.BlockSpec(memory_space=pltpu.SMEM),
            pl.BlockSpec((_ROWS2, 128), lambda i: (i, 0)),
            pl.BlockSpec((_ROWS2, 128), lambda i: (i, 0)),
        ],
        out_specs=pl.BlockSpec((1, 1), lambda i: (0, 0)),
        out_shape=jax.ShapeDtypeStruct((1, 1), jnp.float32),
    )(counts, out_o, out_m)


def kernel(feat, label, groups0, groups1, groups2, W_fco, b_fco, W_fc, b_fc):
    del groups0, groups1, groups2
    lab = label.astype(jnp.int32).reshape(B, 1)
    wt = jnp.zeros((D, 128), jnp.float32)
    wt = wt.at[:, :NR].set(W_fco.T)
    wt = wt.at[:, 64:64 + NR].set(W_fc.T)
    bias = jnp.zeros((1, 128), jnp.float32)
    bias = bias.at[0, :NR].set(b_fco)
    bias = bias.at[0, 64:64 + NR].set(b_fc)

    kr1, kr2 = jax.random.split(jax.random.key(1))
    k1m, k2m = jax.random.split(kr1)
    k1t, k2t = jax.random.split(kr2)
    keys = jnp.concatenate([
        jax.random.key_data(k1m), jax.random.key_data(k2m),
        jax.random.key_data(k1t), jax.random.key_data(k2t),
    ]).astype(jnp.uint32)

    g, cls = _tc1(feat, lab, wt, bias)
    gall, counts = _sca(g, cls.reshape(B))
    idx = _tc0(counts, keys)
    idx3 = idx.reshape(NAUG // 128, 128)
    out_o, out_m = _scb(idx3, gall, counts)
    loss = _tc2(counts, out_o, out_m)
    return loss[0, 0]


# speedup vs baseline: 1.0883x; 1.0883x over previous
import functools

import jax
import jax.numpy as jnp
from jax import lax
from jax.experimental import pallas as pl
from jax.experimental.pallas import tpu as pltpu
from jax.experimental.pallas import tpu_sc as plsc

B = 16384
D = 1024
NR = 51
NEG = -1e30
TBL = B + 8
TRASH = 3 * TBL
NMID = 2 * B
NTAIL = 3 * B
NAUG = NMID + NTAIL

_ROWS1 = 1024


def _tc1_body(x_ref, lab_ref, w_ref, bias_ref, g_ref, cls_ref):
    x = x_ref[...]
    w = w_ref[...]
    p = lax.dot_general(x, w, (((1,), (0,)), ((), ())),
                        preferred_element_type=jnp.float32)
    p = p + bias_ref[...]
    lane = lax.broadcasted_iota(jnp.int32, (_ROWS1, 128), 1)
    valid = lane < NR
    lm = jnp.where(valid, p, NEG)
    mx = jnp.max(lm, axis=1, keepdims=True)
    e = jnp.where(valid, jnp.exp(p - mx), 0.0)
    s = jnp.sum(e, axis=1, keepdims=True)
    prob = e / s
    pmax = jnp.max(prob, axis=1, keepdims=True)
    top = jnp.max(jnp.where((prob == pmax) & valid, lane, -1),
                  axis=1, keepdims=True)
    labv = lab_ref[...]
    labeled = labv != 0
    midf = (top >= 17) & (top < 34) & (pmax > 0.5) & (~labeled)
    tailf = (top >= 34) & (pmax > 0.3) & (~labeled)
    cls = jnp.where(labeled, 1, jnp.where(midf, 2, jnp.where(tailf, 3, 0)))
    g = jnp.where(lane >= 64 + NR, NEG, p)
    lab_f = lax.bitcast_convert_type(labv, jnp.float32)
    top_f = lax.bitcast_convert_type(top, jnp.float32)
    g = jnp.where(lane == 124, lab_f, g)
    g = jnp.where(lane == 125, top_f, g)
    g_ref[...] = g
    cls_ref[...] = cls


def _tc1(feat, lab2, wt, bias):
    return pl.pallas_call(
        _tc1_body,
        grid=(B // _ROWS1,),
        in_specs=[
            pl.BlockSpec((_ROWS1, D), lambda i: (i, 0)),
            pl.BlockSpec((_ROWS1, 1), lambda i: (i, 0)),
            pl.BlockSpec((D, 128), lambda i: (0, 0)),
            pl.BlockSpec((1, 128), lambda i: (0, 0)),
        ],
        out_specs=[
            pl.BlockSpec((_ROWS1, 128), lambda i: (i, 0)),
            pl.BlockSpec((_ROWS1, 1), lambda i: (i, 0)),
        ],
        out_shape=[
            jax.ShapeDtypeStruct((B, 128), jnp.float32),
            jax.ShapeDtypeStruct((B, 1), jnp.int32),
        ],
    )(feat, lab2, wt, bias)


_RPT = B // 32


def _vec_scalars16(v):
    return [v[l] for l in range(16)]


def _sca1_body(meta_hbm, tcnt_hbm, meta_v, cvec_v):
    cid = lax.axis_index("c")
    sid = lax.axis_index("s")
    wid = sid * 2 + cid
    iota = lax.broadcasted_iota(jnp.int32, (16,), 0)
    pltpu.sync_copy(meta_hbm.at[pl.ds(wid * _RPT, _RPT)], meta_v)

    def cnt_step(j, carry):
        co, cm, ct = carry
        cls = meta_v[pl.ds(j * 16, 16)]
        for c in _vec_scalars16(cls):
            co = co + (c == 1).astype(jnp.int32)
            cm = cm + (c == 2).astype(jnp.int32)
            ct = ct + (c == 3).astype(jnp.int32)
        return co, cm, ct

    co, cm, ct = lax.fori_loop(0, _RPT // 16, cnt_step, (0, 0, 0))
    cvec = jnp.where(iota == 0, co,
                     jnp.where(iota == 1, cm,
                               jnp.where(iota == 2, ct, 0)))
    cvec_v[pl.ds(0, 16)] = cvec
    pltpu.sync_copy(cvec_v, tcnt_hbm.at[wid])


def _sca1(meta):
    mesh = plsc.VectorSubcoreMesh(core_axis_name="c", subcore_axis_name="s")
    f = pl.kernel(
        _sca1_body,
        out_type=jax.ShapeDtypeStruct((32, 128), jnp.int32),
        mesh=mesh,
        scratch_types=[
            pltpu.VMEM((_RPT,), jnp.int32),
            pltpu.VMEM((128,), jnp.int32),
        ],
    )
    return f(meta)


def _sca2_body(g_hbm, meta_hbm, tcnt_hbm, gall_hbm, cnt_hbm,
               rows_v, meta_v, cvec_v, allc_v,
               io0, io1, io2, io3, sem):
    io_refs = [io0, io1, io2, io3]
    cid = lax.axis_index("c")
    sid = lax.axis_index("s")
    wid = sid * 2 + cid
    iota = lax.broadcasted_iota(jnp.int32, (16,), 0)
    pltpu.sync_copy(meta_hbm.at[pl.ds(wid * _RPT, _RPT)], meta_v)
    pltpu.sync_copy(tcnt_hbm, allc_v)

    bo, bm, bt = 0, 0, 0
    to, tm, tt = 0, 0, 0
    for t in range(32):
        v = allc_v[t, pl.ds(0, 16)]
        o, m, tl = v[0], v[1], v[2]
        pred = (t < wid).astype(jnp.int32)
        bo = bo + pred * o
        bm = bm + pred * m
        bt = bt + pred * tl
        to = to + o
        tm = tm + m
        tt = tt + tl

    descs = [None, None]
    for ch in range(_RPT // 128):
        def jj_step(jj, carry):
            bo, bm, bt = carry
            cls = meta_v[pl.ds(ch * 128 + jj * 16, 16)]
            dvec = jnp.full((16,), TRASH, jnp.int32)
            for l, c in enumerate(_vec_scalars16(cls)):
                iso = (c == 1).astype(jnp.int32)
                ism = (c == 2).astype(jnp.int32)
                ist = (c == 3).astype(jnp.int32)
                dest = jnp.where(
                    c == 1, bo,
                    jnp.where(c == 2, TBL + bm,
                              jnp.where(c == 3, 2 * TBL + bt, TRASH)))
                dvec = jnp.where(iota == l, dest, dvec)
                bo = bo + iso
                bm = bm + ism
                bt = bt + ist
            io_refs[ch][pl.ds(jj * 16, 16)] = dvec
            return bo, bm, bt

        bo, bm, bt = lax.fori_loop(0, 8, jj_step, (bo, bm, bt))
        buf = ch % 2
        if descs[buf] is not None:
            descs[buf].wait()
        pltpu.sync_copy(g_hbm.at[pl.ds(wid * _RPT + ch * 128, 128)],
                        rows_v.at[buf])
        descs[buf] = pltpu.async_copy(
            rows_v.at[buf], gall_hbm.at[io_refs[ch]], sem)
    for dsc in descs:
        if dsc is not None:
            dsc.wait()

    @pl.when(wid == 0)
    def _emit_counts():
        cvec_v[pl.ds(0, 16)] = jnp.where(
            iota == 0, to,
            jnp.where(iota == 1, tm, jnp.where(iota == 2, tt, 0)))
        pltpu.sync_copy(cvec_v.at[pl.ds(0, 16)], cnt_hbm)


def _sca2(g, meta, tcnt):
    mesh = plsc.VectorSubcoreMesh(core_axis_name="c", subcore_axis_name="s")
    f = pl.kernel(
        _sca2_body,
        out_type=(
            jax.ShapeDtypeStruct((3 * TBL + 8, 128), jnp.float32),
            jax.ShapeDtypeStruct((16,), jnp.int32),
        ),
        mesh=mesh,
        scratch_types=[
            pltpu.VMEM((2, 128, 128), jnp.float32),
            pltpu.VMEM((_RPT,), jnp.int32),
            pltpu.VMEM((128,), jnp.int32),
            pltpu.VMEM((32, 128), jnp.int32),
        ] + [pltpu.VMEM((128,), jnp.int32) for _ in range(4)] + [
            pltpu.SemaphoreType.DMA,
        ],
    )
    return f(g, meta, tcnt)


def _sca(g, meta):
    tcnt = _sca1(meta)
    return _sca2(g, meta, tcnt)


def _rotl(x, d):
    return (x << jnp.uint32(d)) | (x >> jnp.uint32(32 - d))


def _threefry(k0, k1, x0, x1):
    rotations = ((13, 15, 26, 6), (17, 29, 16, 24))
    ks2 = k0 ^ k1 ^ jnp.uint32(0x1BD11BDA)
    ks = (k0, k1, ks2)
    x = [x0 + ks[0], x1 + ks[1]]
    for i in range(5):
        for r in rotations[i % 2]:
            x[0] = x[0] + x[1]
            x[1] = _rotl(x[1], r)
            x[1] = x[0] ^ x[1]
        x[0] = x[0] + ks[(i + 1) % 3]
        x[1] = x[1] + ks[(i + 2) % 3] + jnp.uint32(i + 1)
    return x[0], x[1]


def _tc0_body(cnt_ref, key_ref, idx_ref, *, partitionable):
    n_o = cnt_ref[0]
    span = jnp.maximum(n_o, 1).astype(jnp.uint32)
    m_mid = cnt_ref[1]
    m_tail = cnt_ref[2]
    r = lax.broadcasted_iota(jnp.int32, (NAUG // 128, 128), 0)
    c = lax.broadcasted_iota(jnp.int32, (NAUG // 128, 128), 1)
    flat = r * 128 + c
    part = flat < NMID
    i_in = jnp.where(part, flat, flat - NMID).astype(jnp.uint32)

    def keysel(a, b):
        return jnp.where(part, key_ref[a], key_ref[b])

    def bits(k0, k1):
        if partitionable:
            y0, y1 = _threefry(k0, k1, jnp.zeros_like(i_in), i_in)
            return y0 ^ y1
        n32 = jnp.where(part, 2 * m_mid, 3 * m_tail).astype(jnp.uint32)
        half = (n32 + jnp.uint32(1)) // jnp.uint32(2)
        hi_lane = i_in >= half
        j = jnp.where(hi_lane, i_in - half, i_in)
        x0 = j
        x1 = j + half
        x1 = jnp.where(x1 < n32, x1, jnp.uint32(0))
        y0, y1 = _threefry(k0, k1, x0, x1)
        return jnp.where(hi_lane, y1, y0)

    hi = bits(keysel(0, 4), keysel(1, 5))
    lo = bits(keysel(2, 6), keysel(3, 7))
    mult = jnp.remainder(jnp.uint32(2 ** 16), span)
    mult = jnp.remainder(mult * mult, span)
    off = jnp.remainder(hi, span) * mult + jnp.remainder(lo, span)
    off = jnp.remainder(off, span)
    idx_ref[...] = off.astype(jnp.int32)


def _tc0(counts, keys):
    body = functools.partial(
        _tc0_body, partitionable=bool(jax.config.jax_threefry_partitionable))
    return pl.pallas_call(
        body,
        in_specs=[
            pl.BlockSpec(memory_space=pltpu.SMEM),
            pl.BlockSpec(memory_space=pltpu.SMEM),
        ],
        out_specs=pl.BlockSpec((NAUG // 128, 128), lambda: (0, 0)),
        out_shape=jax.ShapeDtypeStruct((NAUG // 128, 128), jnp.int32),
    )(counts, keys)


_RPW = NAUG // 32
_CH = 128
_NSPL = 4


def _scb_body(idx_hbm, gall_hbm, cnt_hbm, oo_hbm, om_hbm,
              idxo0, idxo1, idxm0, idxm1, obuf, mbuf, cvec_v, semg, semw):
    idxo = [idxo0, idxo1]
    idxm = [idxm0, idxm1]
    cid = lax.axis_index("c")
    sid = lax.axis_index("s")
    nc = 2
    wid = sid * nc + cid
    iota = lax.broadcasted_iota(jnp.int32, (16,), 0)
    pltpu.sync_copy(cnt_hbm, cvec_v)
    cvec = cvec_v[...]
    m_mid = cvec[1]
    m_tail = cvec[2]

    for c in range(_RPW // _CH):
        buf = c % 2
        i0 = wid * _RPW + c * _CH
        gchunk = i0 // _CH
        valid_chunk = jnp.where(i0 < NMID, i0 < 2 * m_mid,
                                i0 - NMID < 3 * m_tail)

        @pl.when(valid_chunk)
        def _do_chunk(i0=i0, gchunk=gchunk, buf=buf):
            pltpu.sync_copy(idx_hbm.at[gchunk], idxo[buf])
            for l in range(8):
                i = i0 + l * 16 + iota
                part = i < NMID
                jm = jnp.where(i < m_mid, i, i - m_mid)
                jm = jnp.where(i < 2 * m_mid, jm, 0)
                it = i - NMID
                jt = jnp.where(it >= m_tail, it - m_tail, it)
                jt = jnp.where(jt >= m_tail, jt - m_tail, jt)
                jt = jnp.where(it < 3 * m_tail, jt, 0)
                j = jnp.where(part, TBL + jm, 2 * TBL + jt)
                idxm[buf][pl.ds(l * 16, 16)] = j
            gd = []
            spl = _CH // _NSPL
            for s in range(_NSPL):
                gd.append(pltpu.async_copy(
                    gall_hbm.at[idxo[buf].at[pl.ds(s * spl, spl)]],
                    obuf.at[buf, pl.ds(s * spl, spl)], semg))
            gd.append(pltpu.async_copy(
                gall_hbm.at[idxm[buf]], mbuf.at[buf], semg))
            for dsc in gd:
                dsc.wait()
            pltpu.sync_copy(obuf.at[buf], oo_hbm.at[pl.ds(i0, _CH)])
            pltpu.sync_copy(mbuf.at[buf], om_hbm.at[pl.ds(i0, _CH)])

    _ = semw


def _scb(idx3, gall, counts):
    mesh = plsc.VectorSubcoreMesh(core_axis_name="c", subcore_axis_name="s")
    f = pl.kernel(
        _scb_body,
        out_type=(
            jax.ShapeDtypeStruct((NAUG, 128), jnp.float32),
            jax.ShapeDtypeStruct((NAUG, 128), jnp.float32),
        ),
        mesh=mesh,
        scratch_types=[
            pltpu.VMEM((128,), jnp.int32),
            pltpu.VMEM((128,), jnp.int32),
            pltpu.VMEM((128,), jnp.int32),
            pltpu.VMEM((128,), jnp.int32),
            pltpu.VMEM((2, _CH, 128), jnp.float32),
            pltpu.VMEM((2, _CH, 128), jnp.float32),
            pltpu.VMEM((16,), jnp.int32),
            pltpu.SemaphoreType.DMA,
            pltpu.SemaphoreType.DMA,
        ],
    )
    return f(idx3, gall, counts)


_ROWS2 = 2048
_NBLK2 = NAUG // _ROWS2


def _tc2_body(cnt_ref, o_ref, m_ref, out_ref):
    pid = pl.program_id(0)
    m_mid = cnt_ref[1]
    m_tail = cnt_ref[2]
    o = o_ref[...]
    m = m_ref[...]
    olab = lax.bitcast_convert_type(o[:, 124:125], jnp.int32)
    mlab = lax.bitcast_convert_type(m[:, 125:126], jnp.int32)
    lane = lax.broadcasted_iota(jnp.int32, (_ROWS2, 128), 1)
    valid = (lane >= 64) & (lane < 64 + NR)
    lg = 0.7 * o + 0.3 * m
    lm = jnp.where(valid, lg, NEG)
    mx = jnp.max(lm, axis=1, keepdims=True)
    e = jnp.where(valid, jnp.exp(lg - mx), 0.0)
    s = jnp.sum(e, axis=1, keepdims=True)
    lse = mx + jnp.log(s)
    pick = (0.7 * jnp.sum(jnp.where(lane - 64 == olab, lg, 0.0), axis=1,
                          keepdims=True)
            + 0.3 * jnp.sum(jnp.where(lane - 64 == mlab, lg, 0.0), axis=1,
                            keepdims=True))
    rl = lse - pick
    rowid = pid * _ROWS2 + lax.broadcasted_iota(jnp.int32, (_ROWS2, 1), 0)
    limit = jnp.where(rowid < NMID, 2 * m_mid, NMID + 3 * m_tail)
    part_sum = jnp.sum(jnp.where(rowid < limit, rl, 0.0))

    @pl.when(pid == 0)
    def _init():
        out_ref[...] = jnp.zeros((1, 1), jnp.float32)

    out_ref[...] = out_ref[...] + part_sum

    @pl.when(pid == _NBLK2 - 1)
    def _fin():
        total = (2 * m_mid + 3 * m_tail).astype(jnp.float32)
        out_ref[...] = jnp.where(total > 0, out_ref[...] / total,
                                 jnp.zeros((1, 1), jnp.float32))


def _tc2(counts, out_o, out_m):
    return pl.pallas_call(
        _tc2_body,
        grid=(_NBLK2,),
        in_specs=[
            pl.BlockSpec(memory_space=pltpu.SMEM),
            pl.BlockSpec((_ROWS2, 128), lambda i: (i, 0)),
            pl.BlockSpec((_ROWS2, 128), lambda i: (i, 0)),
        ],
        out_specs=pl.BlockSpec((1, 1), lambda i: (0, 0)),
        out_shape=jax.ShapeDtypeStruct((1, 1), jnp.float32),
    )(counts, out_o, out_m)


def kernel(feat, label, groups0, groups1, groups2, W_fco, b_fco, W_fc, b_fc):
    del groups0, groups1, groups2
    lab = label.astype(jnp.int32).reshape(B, 1)
    wt = jnp.zeros((D, 128), jnp.float32)
    wt = wt.at[:, :NR].set(W_fco.T)
    wt = wt.at[:, 64:64 + NR].set(W_fc.T)
    bias = jnp.zeros((1, 128), jnp.float32)
    bias = bias.at[0, :NR].set(b_fco)
    bias = bias.at[0, 64:64 + NR].set(b_fc)

    kr1, kr2 = jax.random.split(jax.random.key(1))
    k1m, k2m = jax.random.split(kr1)
    k1t, k2t = jax.random.split(kr2)
    keys = jnp.concatenate([
        jax.random.key_data(k1m), jax.random.key_data(k2m),
        jax.random.key_data(k1t), jax.random.key_data(k2t),
    ]).astype(jnp.uint32)

    g, cls = _tc1(feat, lab, wt, bias)
    gall, counts = _sca(g, cls.reshape(B))
    idx = _tc0(counts, keys)
    idx3 = idx.reshape(NAUG // 128, 128)
    out_o, out_m = _scb(idx3, gall, counts)
    loss = _tc2(counts, out_o, out_m)
    return loss[0, 0]
